# Initial kernel scaffold; baseline (speedup 1.0000x reference)
import jax
import jax.numpy as jnp
from jax.experimental import pallas as pl


def kernel(x, W1, g1, b1, W2, g2, b2, W3, g3, b3, W4, g4, b4, Wc, gc, bc, W5, g5, b5, W6, g6, b6, W7, g7, b7, W8, b8):
    raise NotImplementedError("write your pallas kernel here")


# speedup vs baseline: 4.1505x; 4.1505x over previous
import functools

import jax
import jax.numpy as jnp
from jax.experimental import pallas as pl
from jax.experimental.pallas import tpu as pltpu

KNN = 20
EPS = 1e-5
NEG = -3.0e38


def _lrelu(v):
    return jnp.where(v >= 0, v, 0.2 * v)


def _topk_body(Co, blk, N, xf_ref, xb_ref, w_ref,
               mx_ref, mn_ref, s1_ref, s2_ref):
    xf = xf_ref[...]
    xb = xb_ref[...]
    w = w_ref[...]
    mm = jax.lax.dot_general(
        xb, xf, (((0,), (0,)), ((), ())),
        preferred_element_type=jnp.float32,
        precision=jax.lax.Precision.DEFAULT)
    inner = -2.0 * mm
    xx = jnp.sum(xf * xf, axis=0, keepdims=True)
    xxb = jnp.sum(xb * xb, axis=0, keepdims=True)
    xxcol = jax.lax.transpose(xxb, (1, 0))
    d = (-xxcol) - inner - xx
    iota = jax.lax.broadcasted_iota(jnp.int32, (blk, N), 1)

    h1 = xf.astype(jnp.bfloat16)
    r1 = xf - h1.astype(jnp.float32)
    h2 = r1.astype(jnp.bfloat16)
    h3 = (r1 - h2.astype(jnp.float32)).astype(jnp.bfloat16)

    def step(_, carry):
        d, mx, mn, s1, s2 = carry
        m = jnp.max(d, axis=1, keepdims=True)
        cand = jnp.where(d >= m, iota, N)
        jmin = jnp.min(cand, axis=1, keepdims=True)
        oh = iota == jmin
        d = jnp.where(oh, NEG, d)
        ohb = oh.astype(jnp.bfloat16)
        dims = (((1,), (1,)), ((), ()))
        gx = jax.lax.dot_general(h1, ohb, dims,
                                 preferred_element_type=jnp.float32)
        gx = gx + jax.lax.dot_general(h2, ohb, dims,
                                      preferred_element_type=jnp.float32)
        gx = gx + jax.lax.dot_general(h3, ohb, dims,
                                      preferred_element_type=jnp.float32)
        fd = gx - xb
        y = jax.lax.dot_general(
            w, jnp.concatenate([fd, xb], axis=0), (((1,), (0,)), ((), ())),
            preferred_element_type=jnp.float32,
            precision=jax.lax.Precision.DEFAULT)
        return (d, jnp.maximum(mx, y), jnp.minimum(mn, y), s1 + y, s2 + y * y)

    zeros = jnp.zeros((Co, blk), jnp.float32)
    init = (d, jnp.full((Co, blk), NEG, jnp.float32),
            jnp.full((Co, blk), -NEG, jnp.float32), zeros, zeros)
    _, mx, mn, s1, s2 = jax.lax.fori_loop(0, KNN, step, init)
    mx_ref[...] = mx
    mn_ref[...] = mn
    s1_ref[...] = s1
    s2_ref[...] = s2


def _topk_reduce(x, w, blk=256):
    B, C, N = x.shape
    Co = w.shape[0]
    body = functools.partial(_topk_body, Co, blk, N)
    outs = pl.pallas_call(
        body,
        grid=(B, N // blk),
        in_specs=[
            pl.BlockSpec((None, C, N), lambda b, nb: (b, 0, 0)),
            pl.BlockSpec((None, C, blk), lambda b, nb: (b, 0, nb)),
            pl.BlockSpec((Co, 2 * C), lambda b, nb: (0, 0)),
        ],
        out_specs=[
            pl.BlockSpec((None, Co, blk), lambda b, nb: (b, 0, nb))
            for _ in range(4)
        ],
        out_shape=[
            jax.ShapeDtypeStruct((B, Co, N), jnp.float32) for _ in range(4)
        ],
    )(x, x, w)
    return outs


def _apply_body(B, N, mx_ref, mn_ref, s1_ref, s2_ref, g_ref, b_ref,
                o_ref):
    cnt = float(B * N * KNN)
    s1t = s1_ref[0] + s1_ref[1]
    s2t = s2_ref[0] + s2_ref[1]
    tot1 = jnp.sum(s1t, axis=1, keepdims=True)
    tot2 = jnp.sum(s2t, axis=1, keepdims=True)
    mean = tot1 / cnt
    e2 = tot2 / cnt
    var = e2 - mean * mean
    g = g_ref[...]
    pick = jnp.where(g[None] >= 0, mx_ref[...], mn_ref[...])
    den = jnp.sqrt(var + EPS)
    o_ref[...] = _lrelu((pick - mean[None]) / den[None] * g[None]
                        + b_ref[...][None])


def _bn_apply(mx, mn, s1, s2, g, b):
    B, Co, N = mx.shape
    body = functools.partial(_apply_body, B, N)
    full = pl.BlockSpec((B, Co, N), lambda: (0, 0, 0))
    return pl.pallas_call(
        body,
        grid=(),
        in_specs=[full, full, full, full,
                  pl.BlockSpec((Co, 1), lambda: (0, 0)),
                  pl.BlockSpec((Co, 1), lambda: (0, 0))],
        out_specs=full,
        out_shape=jax.ShapeDtypeStruct((B, Co, N), jnp.float32),
    )(mx, mn, s1, s2, g.reshape(Co, 1), b.reshape(Co, 1))


def _edge_conv(x, W, g, b):
    mx, mn, s1, s2 = _topk_reduce(x, W)
    return _bn_apply(mx, mn, s1, s2, g, b)


def _color_body(B, rgb_ref, wc_ref, g_ref, b_ref, o_ref):
    wc = wc_ref[...]
    craw = [jax.lax.dot_general(wc, rgb_ref[i], (((1,), (0,)), ((), ())),
                                preferred_element_type=jnp.float32, precision=jax.lax.Precision.DEFAULT)
            for i in range(B)]
    n = craw[0].shape[1]
    cnt = float(B * n)
    tot = sum(jnp.sum(cb, axis=1, keepdims=True) for cb in craw)
    tot2 = sum(jnp.sum(cb * cb, axis=1, keepdims=True) for cb in craw)
    mean = tot / cnt
    var = tot2 / cnt - mean * mean
    s = jax.lax.rsqrt(var + EPS) * g_ref[...]
    c = b_ref[...] - mean * s
    for i in range(B):
        o_ref[i] = _lrelu(craw[i] * s + c)


def _color(rgb, wc, g, b):
    B, _, N = rgb.shape
    Co = wc.shape[0]
    body = functools.partial(_color_body, B)
    return pl.pallas_call(
        body,
        grid=(),
        in_specs=[pl.BlockSpec((B, 3, N), lambda: (0, 0, 0)),
                  pl.BlockSpec((Co, 3), lambda: (0, 0)),
                  pl.BlockSpec((Co, 1), lambda: (0, 0)),
                  pl.BlockSpec((Co, 1), lambda: (0, 0))],
        out_specs=pl.BlockSpec((B, Co, N), lambda: (0, 0, 0)),
        out_shape=jax.ShapeDtypeStruct((B, Co, N), jnp.float32),
    )(rgb, wc, g.reshape(Co, 1), b.reshape(Co, 1))


def _mm_stats_body(xin_ref, w_ref, z_ref, a1_ref, a2_ref):
    z = jax.lax.dot_general(
        w_ref[...], xin_ref[...], (((1,), (0,)), ((), ())),
        preferred_element_type=jnp.float32, precision=jax.lax.Precision.DEFAULT)
    z_ref[...] = z
    first = (pl.program_id(0) == 0) & (pl.program_id(1) == 0)
    ps = jnp.sum(z, axis=1, keepdims=True)
    ps2 = jnp.sum(z * z, axis=1, keepdims=True)

    @pl.when(first)
    def _():
        a1_ref[...] = ps
        a2_ref[...] = ps2

    @pl.when(jnp.logical_not(first))
    def _():
        a1_ref[...] += ps
        a2_ref[...] += ps2


def _mm_stats(xin, w, nblk=512):
    B, Ci, N = xin.shape
    Co = w.shape[0]
    return pl.pallas_call(
        _mm_stats_body,
        grid=(B, N // nblk),
        in_specs=[
            pl.BlockSpec((None, Ci, nblk), lambda b, nb: (b, 0, nb)),
            pl.BlockSpec((Co, Ci), lambda b, nb: (0, 0)),
        ],
        out_specs=[
            pl.BlockSpec((None, Co, nblk), lambda b, nb: (b, 0, nb)),
            pl.BlockSpec((Co, 1), lambda b, nb: (0, 0)),
            pl.BlockSpec((Co, 1), lambda b, nb: (0, 0)),
        ],
        out_shape=[
            jax.ShapeDtypeStruct((B, Co, N), jnp.float32),
            jax.ShapeDtypeStruct((Co, 1), jnp.float32),
            jax.ShapeDtypeStruct((Co, 1), jnp.float32),
        ],
    )(xin, w)


def _stats_to_affine(a1, a2, g, b, cnt):
    mean = a1 / cnt
    var = a2 / cnt - mean * mean
    s = jax.lax.rsqrt(var + EPS) * g.reshape(-1, 1)
    c = b.reshape(-1, 1) - mean * s
    return s, c


def _h2_body(xcat_ref, y5_ref, s5_ref, c5_ref, w6a_ref, w6b_ref,
             x5_ref, z6_ref, a1_ref, a2_ref):
    y5n = _lrelu(y5_ref[...] * s5_ref[...] + c5_ref[...])
    x5_ref[...] = y5n
    z = jax.lax.dot_general(
        w6a_ref[...], xcat_ref[...], (((1,), (0,)), ((), ())),
        preferred_element_type=jnp.float32, precision=jax.lax.Precision.DEFAULT)
    z = z + jax.lax.dot_general(
        w6b_ref[...], y5n, (((1,), (0,)), ((), ())),
        preferred_element_type=jnp.float32, precision=jax.lax.Precision.DEFAULT)
    z6_ref[...] = z
    first = (pl.program_id(0) == 0) & (pl.program_id(1) == 0)
    ps = jnp.sum(z, axis=1, keepdims=True)
    ps2 = jnp.sum(z * z, axis=1, keepdims=True)

    @pl.when(first)
    def _():
        a1_ref[...] = ps
        a2_ref[...] = ps2

    @pl.when(jnp.logical_not(first))
    def _():
        a1_ref[...] += ps
        a2_ref[...] += ps2


def _h2(xcat, y5raw, s5, c5, w6a, w6b, nblk=512):
    B, _, N = xcat.shape
    C5 = y5raw.shape[1]
    Co = w6a.shape[0]
    return pl.pallas_call(
        _h2_body,
        grid=(B, N // nblk),
        in_specs=[
            pl.BlockSpec((None, xcat.shape[1], nblk), lambda b, nb: (b, 0, nb)),
            pl.BlockSpec((None, C5, nblk), lambda b, nb: (b, 0, nb)),
            pl.BlockSpec((C5, 1), lambda b, nb: (0, 0)),
            pl.BlockSpec((C5, 1), lambda b, nb: (0, 0)),
            pl.BlockSpec((Co, w6a.shape[1]), lambda b, nb: (0, 0)),
            pl.BlockSpec((Co, C5), lambda b, nb: (0, 0)),
        ],
        out_specs=[
            pl.BlockSpec((None, C5, nblk), lambda b, nb: (b, 0, nb)),
            pl.BlockSpec((None, Co, nblk), lambda b, nb: (b, 0, nb)),
            pl.BlockSpec((Co, 1), lambda b, nb: (0, 0)),
            pl.BlockSpec((Co, 1), lambda b, nb: (0, 0)),
        ],
        out_shape=[
            jax.ShapeDtypeStruct((B, C5, N), jnp.float32),
            jax.ShapeDtypeStruct((B, Co, N), jnp.float32),
            jax.ShapeDtypeStruct((Co, 1), jnp.float32),
            jax.ShapeDtypeStruct((Co, 1), jnp.float32),
        ],
    )(xcat, y5raw, s5, c5, w6a, w6b)


def _h3_body(z_ref, s_ref, c_ref, w_ref, zo_ref, a1_ref, a2_ref):
    zn = _lrelu(z_ref[...] * s_ref[...] + c_ref[...])
    z = jax.lax.dot_general(
        w_ref[...], zn, (((1,), (0,)), ((), ())),
        preferred_element_type=jnp.float32, precision=jax.lax.Precision.DEFAULT)
    zo_ref[...] = z
    first = (pl.program_id(0) == 0) & (pl.program_id(1) == 0)
    ps = jnp.sum(z, axis=1, keepdims=True)
    ps2 = jnp.sum(z * z, axis=1, keepdims=True)

    @pl.when(first)
    def _():
        a1_ref[...] = ps
        a2_ref[...] = ps2

    @pl.when(jnp.logical_not(first))
    def _():
        a1_ref[...] += ps
        a2_ref[...] += ps2


def _h3(zin, s, c, w, nblk=512):
    B, Ci, N = zin.shape
    Co = w.shape[0]
    return pl.pallas_call(
        _h3_body,
        grid=(B, N // nblk),
        in_specs=[
            pl.BlockSpec((None, Ci, nblk), lambda b, nb: (b, 0, nb)),
            pl.BlockSpec((Ci, 1), lambda b, nb: (0, 0)),
            pl.BlockSpec((Ci, 1), lambda b, nb: (0, 0)),
            pl.BlockSpec((Co, Ci), lambda b, nb: (0, 0)),
        ],
        out_specs=[
            pl.BlockSpec((None, Co, nblk), lambda b, nb: (b, 0, nb)),
            pl.BlockSpec((Co, 1), lambda b, nb: (0, 0)),
            pl.BlockSpec((Co, 1), lambda b, nb: (0, 0)),
        ],
        out_shape=[
            jax.ShapeDtypeStruct((B, Co, N), jnp.float32),
            jax.ShapeDtypeStruct((Co, 1), jnp.float32),
            jax.ShapeDtypeStruct((Co, 1), jnp.float32),
        ],
    )(zin, s, c, w)


def _h4_body(z_ref, s_ref, c_ref, w_ref, b8_ref, o_ref):
    zn = _lrelu(z_ref[...] * s_ref[...] + c_ref[...])
    o_ref[...] = jax.lax.dot_general(
        w_ref[...], zn, (((1,), (0,)), ((), ())),
        preferred_element_type=jnp.float32, precision=jax.lax.Precision.DEFAULT) + b8_ref[...]


def _h4(zin, s, c, w, b8, nblk=512):
    B, Ci, N = zin.shape
    Co = w.shape[0]
    return pl.pallas_call(
        _h4_body,
        grid=(B, N // nblk),
        in_specs=[
            pl.BlockSpec((None, Ci, nblk), lambda b, nb: (b, 0, nb)),
            pl.BlockSpec((Ci, 1), lambda b, nb: (0, 0)),
            pl.BlockSpec((Ci, 1), lambda b, nb: (0, 0)),
            pl.BlockSpec((Co, Ci), lambda b, nb: (0, 0)),
            pl.BlockSpec((Co, 1), lambda b, nb: (0, 0)),
        ],
        out_specs=pl.BlockSpec((None, Co, nblk), lambda b, nb: (b, 0, nb)),
        out_shape=jax.ShapeDtypeStruct((B, Co, N), jnp.float32),
    )(zin, s, c, w, b8.reshape(Co, 1))


def kernel(x, W1, g1, b1, W2, g2, b2, W3, g3, b3, W4, g4, b4, Wc, gc, bc,
           W5, g5, b5, W6, g6, b6, W7, g7, b7, W8, b8):
    B, _, N = x.shape
    xyz = x[:, :3, :]
    rgb = x[:, 3:6, :]
    x1 = _edge_conv(xyz, W1, g1, b1)
    x2 = _edge_conv(x1, W2, g2, b2)
    x3 = _edge_conv(x2, W3, g3, b3)
    x4 = _edge_conv(x3, W4, g4, b4)
    col = _color(rgb, Wc, gc, bc)
    xcat = jnp.concatenate([x1, x2, x3, x4, col], axis=1)
    cnt = float(B * N)

    y5raw, a1, a2 = _mm_stats(xcat, W5)
    s5, c5 = _stats_to_affine(a1, a2, g5, b5, cnt)
    x5, z6raw, a1, a2 = _h2(xcat, y5raw, s5, c5, W6[:, :384], W6[:, 384:])
    s6, c6 = _stats_to_affine(a1, a2, g6, b6, cnt)
    z7raw, a1, a2 = _h3(z6raw, s6, c6, W7)
    s7, c7 = _stats_to_affine(a1, a2, g7, b7, cnt)
    logits = _h4(z7raw, s7, c7, W8, b8)
    return (jnp.transpose(logits, (0, 2, 1)), x5)
